# Initial kernel scaffold; baseline (speedup 1.0000x reference)
import functools

import jax
import jax.numpy as jnp
from jax import lax
from jax.experimental import pallas as pl
from jax.experimental.pallas import tpu as pltpu
from jax.experimental.pallas import tpu_sc as plsc

N = 10000
E = 320000
D = 128
B = 16
SV = 64
A = 32

NC = 2
NS = 16
NT = NC * NS
EPT = E // NT
K = 80
NCH = EPT // K
RPT = N // NS

_mesh = plsc.VectorSubcoreMesh(
    core_axis_name="c", subcore_axis_name="s", num_cores=NC, num_subcores=NS)


@functools.partial(
    pl.kernel,
    out_type=jax.ShapeDtypeStruct((NC, N, 16), jnp.float32),
    mesh=_mesh,
    scratch_types=[
        pltpu.VMEM((NCH, K), jnp.int32),
        pltpu.VMEM((K, 16), jnp.float32),
        pltpu.VMEM_SHARED((N, 16), jnp.float32),
    ],
)
def _deg_kernel(dst3_hbm, ones_hbm, zeros16_hbm, out_hbm, dstv, onesv, acc):
    c = lax.axis_index("c")
    s = lax.axis_index("s")
    t = c * NS + s
    pltpu.sync_copy(dst3_hbm.at[t], dstv)
    pltpu.sync_copy(ones_hbm, onesv)
    pltpu.sync_copy(zeros16_hbm, acc.at[pl.ds(s * RPT, RPT)])
    plsc.subcore_barrier()

    def body(j, carry):
        pltpu.sync_copy(onesv, acc.at[dstv.at[j]], add=True)
        return carry

    lax.fori_loop(0, NCH, body, 0)
    plsc.subcore_barrier()
    pltpu.sync_copy(acc.at[pl.ds(s * RPT, RPT)],
                    out_hbm.at[c, pl.ds(s * RPT, RPT)])


@functools.partial(
    pl.kernel,
    out_type=jax.ShapeDtypeStruct((NC, N, D), jnp.float32),
    mesh=_mesh,
    scratch_types=[
        pltpu.VMEM((NCH, K), jnp.int32),
        pltpu.VMEM((NCH, K), jnp.int32),
        pltpu.VMEM((K, D), jnp.float32),
        pltpu.VMEM_SHARED((N, D), jnp.float32),
        pltpu.SemaphoreType.DMA,
    ],
)
def _prop_kernel(h_hbm, src3_hbm, dst3_hbm, zeros_hbm, out_hbm,
                 srcv, dstv, rows, acc, sem):
    c = lax.axis_index("c")
    s = lax.axis_index("s")
    t = c * NS + s
    pltpu.sync_copy(src3_hbm.at[t], srcv)
    pltpu.sync_copy(dst3_hbm.at[t], dstv)
    pltpu.sync_copy(zeros_hbm, acc.at[pl.ds(s * RPT, RPT)])
    plsc.subcore_barrier()

    def body(j, carry):
        pltpu.async_copy(h_hbm.at[srcv.at[j]], rows, sem).wait()
        pltpu.sync_copy(rows, acc.at[dstv.at[j]], add=True)
        return carry

    lax.fori_loop(0, NCH, body, 0)
    plsc.subcore_barrier()
    pltpu.sync_copy(acc.at[pl.ds(s * RPT, RPT)],
                    out_hbm.at[c, pl.ds(s * RPT, RPT)])


R = 1000
G = N // R


def _dis_from(dacc_ref):
    deg = dacc_ref[0, :, 0:1] + dacc_ref[1, :, 0:1] + 1.0
    return lax.rsqrt(deg)


def _mm1_body(x_ref, w1_ref, dacc_ref, out_ref):
    h = jnp.dot(x_ref[...], w1_ref[...], preferred_element_type=jnp.float32)
    out_ref[...] = h * _dis_from(dacc_ref)


def _mid_body(acc_ref, h1_ref, dacc_ref, b1_ref, w2_ref, out_ref):
    dis = _dis_from(dacc_ref)
    s1 = jnp.maximum(
        (acc_ref[0] + acc_ref[1] + h1_ref[...]) * dis + b1_ref[...], 0.0)
    h2 = jnp.dot(s1, w2_ref[...], preferred_element_type=jnp.float32)
    out_ref[...] = h2 * dis


def _head_body(acc_ref, h2_ref, dacc_ref, b2_ref, batch_ref, sv_ref,
               wg_ref, bg_ref, wf_ref, bf_ref, out_ref, pooled_acc, cnt_acc):
    i = pl.program_id(0)

    @pl.when(i == 0)
    def _():
        pooled_acc[...] = jnp.zeros_like(pooled_acc)
        cnt_acc[...] = jnp.zeros_like(cnt_acc)

    dis = _dis_from(dacc_ref)
    s2 = jnp.maximum(
        (acc_ref[0] + acc_ref[1] + h2_ref[...]) * dis + b2_ref[...], 0.0)
    gids = lax.broadcasted_iota(jnp.int32, (B, R), 0)
    m = (batch_ref[0:1, :] == gids).astype(jnp.float32)
    pooled_acc[...] += jnp.dot(m, s2, preferred_element_type=jnp.float32)
    cnt_acc[...] += jnp.sum(m, axis=1, keepdims=True)

    @pl.when(i == G - 1)
    def _():
        pooled = pooled_acc[...] / jnp.maximum(cnt_acc[...], 1.0)
        z = jnp.maximum(
            jnp.dot(pooled, wg_ref[0:D, :], preferred_element_type=jnp.float32)
            + jnp.dot(sv_ref[...], wg_ref[D:D + SV, :],
                      preferred_element_type=jnp.float32)
            + bg_ref[...], 0.0)
        out_ref[...] = jnp.tanh(
            jnp.dot(z, wf_ref[...], preferred_element_type=jnp.float32)
            + bf_ref[...])


def _full(shape):
    return pl.BlockSpec(shape, lambda i: tuple(0 for _ in shape))


def _rows(shape, dim=0):
    def idx(i):
        out = [0] * len(shape)
        out[dim] = i
        return tuple(out)
    return pl.BlockSpec(shape, idx)


def kernel(x, edge_index, batch, state_vector, W1, b1, W2, b2, Wg, bg, Wf, bf):
    src3 = edge_index[0].reshape(NT, NCH, K)
    dst3 = edge_index[1].reshape(NT, NCH, K)
    ones16 = jnp.ones((K, 16), jnp.float32)
    zeros16 = jnp.zeros((RPT, 16), jnp.float32)
    zeros = jnp.zeros((RPT, D), jnp.float32)
    batch2 = batch.reshape(1, N)
    b1r = b1.reshape(1, D)
    b2r = b2.reshape(1, D)
    bgr = bg.reshape(1, 256)
    bfr = bf.reshape(1, A)

    dacc = _deg_kernel(dst3, ones16, zeros16)

    h1s = pl.pallas_call(
        _mm1_body,
        grid=(G,),
        in_specs=[_rows((R, D)), _full((D, D)), _rows((NC, R, 16), dim=1)],
        out_specs=_rows((R, D)),
        out_shape=jax.ShapeDtypeStruct((N, D), jnp.float32),
    )(x, W1, dacc)

    acc1 = _prop_kernel(h1s, src3, dst3, zeros)

    h2s = pl.pallas_call(
        _mid_body,
        grid=(G,),
        in_specs=[_rows((NC, R, D), dim=1), _rows((R, D)),
                  _rows((NC, R, 16), dim=1), _full((1, D)), _full((D, D))],
        out_specs=_rows((R, D)),
        out_shape=jax.ShapeDtypeStruct((N, D), jnp.float32),
    )(acc1, h1s, dacc, b1r, W2)

    acc2 = _prop_kernel(h2s, src3, dst3, zeros)

    out = pl.pallas_call(
        _head_body,
        grid=(G,),
        in_specs=[_rows((NC, R, D), dim=1), _rows((R, D)),
                  _rows((NC, R, 16), dim=1), _full((1, D)),
                  _rows((1, R), dim=1), _full((B, SV)),
                  _full((D + SV, 256)), _full((1, 256)),
                  _full((256, A)), _full((1, A))],
        out_specs=_full((B, A)),
        out_shape=jax.ShapeDtypeStruct((B, A), jnp.float32),
        scratch_shapes=[pltpu.VMEM((B, D), jnp.float32),
                        pltpu.VMEM((B, D), jnp.float32)],
    )(acc2, h2s, dacc, b2r, batch2, state_vector, Wg, bgr, Wf, bfr)

    return out


# speedup vs baseline: 17.9895x; 17.9895x over previous
import functools

import jax
import jax.numpy as jnp
from jax import lax
from jax.experimental import pallas as pl
from jax.experimental.pallas import tpu as pltpu
from jax.experimental.pallas import tpu_sc as plsc

N = 10000
E = 320000
D = 128
B = 16
SV = 64
A = 32

NC = 2
NS = 16
NT = NC * NS
EPT = E // NT
K = 80
NCH = EPT // K
NP = 10240
RPT = NP // NS

def _deg_body(dst3_hbm, ones_hbm, zeros_hbm, out_hbm, dstv, onesv, acc):
    c = lax.axis_index("c")
    s = lax.axis_index("s")
    t = c * NS + s
    pltpu.sync_copy(dst3_hbm.at[t], dstv)
    pltpu.sync_copy(ones_hbm, onesv)
    pltpu.sync_copy(zeros_hbm, acc.at[pl.ds(s * RPT, RPT)])
    plsc.subcore_barrier()

    def body(j, carry):
        pltpu.sync_copy(onesv, acc.at[dstv.at[j]], add=True)
        return carry

    lax.fori_loop(0, NCH, body, 0)
    plsc.subcore_barrier()
    pltpu.sync_copy(acc.at[pl.ds(s * RPT, RPT)],
                    out_hbm.at[c, pl.ds(s * RPT, RPT)])


def _prop_body(h_hbm, src3_hbm, dst3_hbm, zeros_hbm, out_hbm,
               srcv, dstv, rows, acc, sem):
    c = lax.axis_index("c")
    s = lax.axis_index("s")
    t = c * NS + s
    pltpu.sync_copy(src3_hbm.at[t], srcv)
    pltpu.sync_copy(dst3_hbm.at[t], dstv)
    pltpu.sync_copy(zeros_hbm, acc.at[pl.ds(s * RPT, RPT)])
    plsc.subcore_barrier()

    def body(j, carry):
        pltpu.async_copy(h_hbm.at[srcv.at[j]], rows, sem).wait()
        pltpu.sync_copy(rows, acc.at[dstv.at[j]], add=True)
        return carry

    lax.fori_loop(0, NCH, body, 0)
    plsc.subcore_barrier()
    pltpu.sync_copy(acc.at[pl.ds(s * RPT, RPT)],
                    out_hbm.at[c, pl.ds(s * RPT, RPT)])


@functools.cache
def _sc_kernels():
    mesh = plsc.VectorSubcoreMesh(
        core_axis_name="c", subcore_axis_name="s",
        num_cores=NC, num_subcores=NS)
    deg = pl.kernel(
        _deg_body,
        out_type=jax.ShapeDtypeStruct((NC, NP, D), jnp.float32),
        mesh=mesh,
        scratch_types=[
            pltpu.VMEM((NCH, K), jnp.int32),
            pltpu.VMEM((K, D), jnp.float32),
            pltpu.VMEM_SHARED((NP, D), jnp.float32),
        ],
    )
    prop = pl.kernel(
        _prop_body,
        out_type=jax.ShapeDtypeStruct((NC, NP, D), jnp.float32),
        mesh=mesh,
        scratch_types=[
            pltpu.VMEM((NCH, K), jnp.int32),
            pltpu.VMEM((NCH, K), jnp.int32),
            pltpu.VMEM((K, D), jnp.float32),
            pltpu.VMEM_SHARED((NP, D), jnp.float32),
            pltpu.SemaphoreType.DMA,
        ],
    )
    return deg, prop


R = 1000
G = N // R


def _dis_from(dacc_ref):
    deg = dacc_ref[0, :, 0:1] + dacc_ref[1, :, 0:1] + 1.0
    return lax.rsqrt(deg)


def _mm1_body(x_ref, w1_ref, dacc_ref, out_ref):
    h = jnp.dot(x_ref[...], w1_ref[...], preferred_element_type=jnp.float32)
    out_ref[...] = h * _dis_from(dacc_ref)


def _mid_body(acc_ref, h1_ref, dacc_ref, b1_ref, w2_ref, out_ref):
    dis = _dis_from(dacc_ref)
    s1 = jnp.maximum(
        (acc_ref[0] + acc_ref[1] + h1_ref[...]) * dis + b1_ref[...], 0.0)
    h2 = jnp.dot(s1, w2_ref[...], preferred_element_type=jnp.float32)
    out_ref[...] = h2 * dis


def _head_body(acc_ref, h2_ref, dacc_ref, b2_ref, batch_ref, sv_ref,
               wg_ref, bg_ref, wf_ref, bf_ref, out_ref, pooled_acc, cnt_acc):
    i = pl.program_id(0)

    @pl.when(i == 0)
    def _():
        pooled_acc[...] = jnp.zeros_like(pooled_acc)
        cnt_acc[...] = jnp.zeros_like(cnt_acc)

    dis = _dis_from(dacc_ref)
    s2 = jnp.maximum(
        (acc_ref[0] + acc_ref[1] + h2_ref[...]) * dis + b2_ref[...], 0.0)
    gids = lax.broadcasted_iota(jnp.int32, (B, R), 0)
    m = (batch_ref[0, 0:1, :] == gids).astype(jnp.float32)
    pooled_acc[...] += jnp.dot(m, s2, preferred_element_type=jnp.float32)
    cnt_acc[...] += jnp.sum(m, axis=1, keepdims=True)

    @pl.when(i == G - 1)
    def _():
        pooled = pooled_acc[...] / jnp.maximum(cnt_acc[...], 1.0)
        z = jnp.maximum(
            jnp.dot(pooled, wg_ref[0:D, :], preferred_element_type=jnp.float32)
            + jnp.dot(sv_ref[...], wg_ref[D:D + SV, :],
                      preferred_element_type=jnp.float32)
            + bg_ref[...], 0.0)
        out_ref[...] = jnp.tanh(
            jnp.dot(z, wf_ref[...], preferred_element_type=jnp.float32)
            + bf_ref[...])


def _full(shape):
    return pl.BlockSpec(shape, lambda i: tuple(0 for _ in shape))


def _rows(shape, dim=0):
    def idx(i):
        out = [0] * len(shape)
        out[dim] = i
        return tuple(out)
    return pl.BlockSpec(shape, idx)


_DBG_JNP_DEG = False
_DBG_JNP_PROP = False


def kernel(x, edge_index, batch, state_vector, W1, b1, W2, b2, Wg, bg, Wf, bf):
    src3 = edge_index[0].reshape(NT, NCH, K)
    dst3 = edge_index[1].reshape(NT, NCH, K)
    ones128 = jnp.ones((K, D), jnp.float32)
    zeros = jnp.zeros((RPT, D), jnp.float32)
    batch3 = batch.reshape(G, 1, R)
    b1r = b1.reshape(1, D)
    b2r = b2.reshape(1, D)
    bgr = bg.reshape(1, 256)
    bfr = bf.reshape(1, A)

    _deg_kernel, _prop_kernel = _sc_kernels()
    if _DBG_JNP_DEG:
        dacc = jnp.zeros((NC, NP, D), jnp.float32).at[0, edge_index[1], :].add(1.0)
    else:
        dacc = _deg_kernel(dst3, ones128, zeros)

    h1s = pl.pallas_call(
        _mm1_body,
        grid=(G,),
        in_specs=[_rows((R, D)), _full((D, D)), _rows((NC, R, D), dim=1)],
        out_specs=_rows((R, D)),
        out_shape=jax.ShapeDtypeStruct((N, D), jnp.float32),
    )(x, W1, dacc)

    if _DBG_JNP_PROP:
        acc1 = jnp.zeros((NC, NP, D), jnp.float32).at[0, edge_index[1]].add(h1s[edge_index[0]])
    else:
        acc1 = _prop_kernel(h1s, src3, dst3, zeros)

    h2s = pl.pallas_call(
        _mid_body,
        grid=(G,),
        in_specs=[_rows((NC, R, D), dim=1), _rows((R, D)),
                  _rows((NC, R, D), dim=1), _full((1, D)), _full((D, D))],
        out_specs=_rows((R, D)),
        out_shape=jax.ShapeDtypeStruct((N, D), jnp.float32),
    )(acc1, h1s, dacc, b1r, W2)

    if _DBG_JNP_PROP:
        acc2 = jnp.zeros((NC, NP, D), jnp.float32).at[0, edge_index[1]].add(h2s[edge_index[0]])
    else:
        acc2 = _prop_kernel(h2s, src3, dst3, zeros)

    out = pl.pallas_call(
        _head_body,
        grid=(G,),
        in_specs=[_rows((NC, R, D), dim=1), _rows((R, D)),
                  _rows((NC, R, D), dim=1), _full((1, D)),
                  _rows((1, 1, R), dim=0), _full((B, SV)),
                  _full((D + SV, 256)), _full((1, 256)),
                  _full((256, A)), _full((1, A))],
        out_specs=_full((B, A)),
        out_shape=jax.ShapeDtypeStruct((B, A), jnp.float32),
        scratch_shapes=[pltpu.VMEM((B, D), jnp.float32),
                        pltpu.VMEM((B, D), jnp.float32)],
    )(acc2, h2s, dacc, b2r, batch3, state_vector, Wg, bgr, Wf, bfr)

    return out


# speedup vs baseline: 26.3979x; 1.4674x over previous
import functools

import jax
import jax.numpy as jnp
from jax import lax
from jax.experimental import pallas as pl
from jax.experimental.pallas import tpu as pltpu
from jax.experimental.pallas import tpu_sc as plsc

N = 10000
E = 320000
D = 128
B = 16
SV = 64
A = 32

NC = 2
NS = 16
NT = NC * NS
EPT = E // NT
K = 80
NCH = EPT // K
NP = 10240
RPT = NP // NS

def _deg_body(dst3_hbm, ones_hbm, zeros_hbm, out_hbm, dstv, onesv, acc):
    c = lax.axis_index("c")
    s = lax.axis_index("s")
    t = c * NS + s
    pltpu.sync_copy(dst3_hbm.at[t], dstv)
    pltpu.sync_copy(ones_hbm, onesv)
    pltpu.sync_copy(zeros_hbm, acc.at[pl.ds(s * RPT, RPT)])
    plsc.subcore_barrier()

    def body(j, carry):
        pltpu.sync_copy(onesv, acc.at[dstv.at[j]], add=True)
        return carry

    lax.fori_loop(0, NCH, body, 0)
    plsc.subcore_barrier()
    pltpu.sync_copy(acc.at[pl.ds(s * RPT, RPT)],
                    out_hbm.at[c, pl.ds(s * RPT, RPT)])


def _prop_body(h_hbm, src3_hbm, dst3_hbm, zeros_hbm, out_hbm,
               srcv, dstb0, dstb1, rows0, rows1, acc,
               semg0, semg1, semd0, semd1):
    c = lax.axis_index("c")
    s = lax.axis_index("s")
    t = c * NS + s
    pltpu.sync_copy(src3_hbm.at[t], srcv)
    pltpu.sync_copy(zeros_hbm, acc.at[pl.ds(s * RPT, RPT)])
    plsc.subcore_barrier()

    def issue(j, rows, dstb, semg, semd):
        pltpu.async_copy(dst3_hbm.at[t, j], dstb, semd)
        pltpu.async_copy(h_hbm.at[srcv.at[j]], rows, semg)

    def drain(rows, dstb, semg, semd):
        pltpu.make_async_copy(dst3_hbm.at[t, 0], dstb, semd).wait()
        pltpu.make_async_copy(h_hbm.at[srcv.at[0]], rows, semg).wait()
        pltpu.sync_copy(rows, acc.at[dstb], add=True)

    issue(0, rows0, dstb0, semg0, semd0)
    issue(1, rows1, dstb1, semg1, semd1)

    def body(i, carry):
        j = 2 * i
        drain(rows0, dstb0, semg0, semd0)

        @pl.when(j + 2 < NCH)
        def _():
            issue(j + 2, rows0, dstb0, semg0, semd0)

        drain(rows1, dstb1, semg1, semd1)

        @pl.when(j + 3 < NCH)
        def _():
            issue(j + 3, rows1, dstb1, semg1, semd1)

        return carry

    lax.fori_loop(0, (NCH - 1) // 2, body, 0)
    drain(rows0, dstb0, semg0, semd0)
    plsc.subcore_barrier()
    pltpu.sync_copy(acc.at[pl.ds(s * RPT, RPT)],
                    out_hbm.at[c, pl.ds(s * RPT, RPT)])


@functools.cache
def _sc_kernels():
    mesh = plsc.VectorSubcoreMesh(
        core_axis_name="c", subcore_axis_name="s",
        num_cores=NC, num_subcores=NS)
    deg = pl.kernel(
        _deg_body,
        out_type=jax.ShapeDtypeStruct((NC, NP, D), jnp.float32),
        mesh=mesh,
        scratch_types=[
            pltpu.VMEM((NCH, K), jnp.int32),
            pltpu.VMEM((K, D), jnp.float32),
            pltpu.VMEM_SHARED((NP, D), jnp.float32),
        ],
    )
    prop = pl.kernel(
        _prop_body,
        out_type=jax.ShapeDtypeStruct((NC, NP, D), jnp.float32),
        mesh=mesh,
        scratch_types=[
            pltpu.VMEM((NCH, K), jnp.int32),
            pltpu.VMEM((K,), jnp.int32),
            pltpu.VMEM((K,), jnp.int32),
            pltpu.VMEM((K, D), jnp.float32),
            pltpu.VMEM((K, D), jnp.float32),
            pltpu.VMEM_SHARED((NP, D), jnp.float32),
            pltpu.SemaphoreType.DMA,
            pltpu.SemaphoreType.DMA,
            pltpu.SemaphoreType.DMA,
            pltpu.SemaphoreType.DMA,
        ],
    )
    return deg, prop


R = 1000
G = N // R


def _dis_from(dacc_ref):
    deg = dacc_ref[0, :, 0:1] + dacc_ref[1, :, 0:1] + 1.0
    return lax.rsqrt(deg)


def _mm1_body(x_ref, w1_ref, dacc_ref, out_ref):
    h = jnp.dot(x_ref[...], w1_ref[...], preferred_element_type=jnp.float32)
    out_ref[...] = h * _dis_from(dacc_ref)


def _mid_body(acc_ref, h1_ref, dacc_ref, b1_ref, w2_ref, out_ref):
    dis = _dis_from(dacc_ref)
    s1 = jnp.maximum(
        (acc_ref[0] + acc_ref[1] + h1_ref[...]) * dis + b1_ref[...], 0.0)
    h2 = jnp.dot(s1, w2_ref[...], preferred_element_type=jnp.float32)
    out_ref[...] = h2 * dis


def _head_body(acc_ref, h2_ref, dacc_ref, b2_ref, batch_ref, sv_ref,
               wg_ref, bg_ref, wf_ref, bf_ref, out_ref, pooled_acc, cnt_acc):
    i = pl.program_id(0)

    @pl.when(i == 0)
    def _():
        pooled_acc[...] = jnp.zeros_like(pooled_acc)
        cnt_acc[...] = jnp.zeros_like(cnt_acc)

    dis = _dis_from(dacc_ref)
    s2 = jnp.maximum(
        (acc_ref[0] + acc_ref[1] + h2_ref[...]) * dis + b2_ref[...], 0.0)
    gids = lax.broadcasted_iota(jnp.int32, (B, R), 0)
    m = (batch_ref[0, 0:1, :] == gids).astype(jnp.float32)
    pooled_acc[...] += jnp.dot(m, s2, preferred_element_type=jnp.float32)
    cnt_acc[...] += jnp.sum(m, axis=1, keepdims=True)

    @pl.when(i == G - 1)
    def _():
        pooled = pooled_acc[...] / jnp.maximum(cnt_acc[...], 1.0)
        z = jnp.maximum(
            jnp.dot(pooled, wg_ref[0:D, :], preferred_element_type=jnp.float32)
            + jnp.dot(sv_ref[...], wg_ref[D:D + SV, :],
                      preferred_element_type=jnp.float32)
            + bg_ref[...], 0.0)
        out_ref[...] = jnp.tanh(
            jnp.dot(z, wf_ref[...], preferred_element_type=jnp.float32)
            + bf_ref[...])


def _full(shape):
    return pl.BlockSpec(shape, lambda i: tuple(0 for _ in shape))


def _rows(shape, dim=0):
    def idx(i):
        out = [0] * len(shape)
        out[dim] = i
        return tuple(out)
    return pl.BlockSpec(shape, idx)


_DBG_JNP_DEG = False
_DBG_JNP_PROP = False


def kernel(x, edge_index, batch, state_vector, W1, b1, W2, b2, Wg, bg, Wf, bf):
    src3 = edge_index[0].reshape(NT, NCH, K)
    dst3 = edge_index[1].reshape(NT, NCH, K)
    ones128 = jnp.ones((K, D), jnp.float32)
    zeros = jnp.zeros((RPT, D), jnp.float32)
    batch3 = batch.reshape(G, 1, R)
    b1r = b1.reshape(1, D)
    b2r = b2.reshape(1, D)
    bgr = bg.reshape(1, 256)
    bfr = bf.reshape(1, A)

    _deg_kernel, _prop_kernel = _sc_kernels()
    if _DBG_JNP_DEG:
        dacc = jnp.zeros((NC, NP, D), jnp.float32).at[0, edge_index[1], :].add(1.0)
    else:
        dacc = _deg_kernel(dst3, ones128, zeros)

    h1s = pl.pallas_call(
        _mm1_body,
        grid=(G,),
        in_specs=[_rows((R, D)), _full((D, D)), _rows((NC, R, D), dim=1)],
        out_specs=_rows((R, D)),
        out_shape=jax.ShapeDtypeStruct((N, D), jnp.float32),
    )(x, W1, dacc)

    if _DBG_JNP_PROP:
        acc1 = jnp.zeros((NC, NP, D), jnp.float32).at[0, edge_index[1]].add(h1s[edge_index[0]])
    else:
        acc1 = _prop_kernel(h1s, src3, dst3, zeros)

    h2s = pl.pallas_call(
        _mid_body,
        grid=(G,),
        in_specs=[_rows((NC, R, D), dim=1), _rows((R, D)),
                  _rows((NC, R, D), dim=1), _full((1, D)), _full((D, D))],
        out_specs=_rows((R, D)),
        out_shape=jax.ShapeDtypeStruct((N, D), jnp.float32),
    )(acc1, h1s, dacc, b1r, W2)

    if _DBG_JNP_PROP:
        acc2 = jnp.zeros((NC, NP, D), jnp.float32).at[0, edge_index[1]].add(h2s[edge_index[0]])
    else:
        acc2 = _prop_kernel(h2s, src3, dst3, zeros)

    out = pl.pallas_call(
        _head_body,
        grid=(G,),
        in_specs=[_rows((NC, R, D), dim=1), _rows((R, D)),
                  _rows((NC, R, D), dim=1), _full((1, D)),
                  _rows((1, 1, R), dim=0), _full((B, SV)),
                  _full((D + SV, 256)), _full((1, 256)),
                  _full((256, A)), _full((1, A))],
        out_specs=_full((B, A)),
        out_shape=jax.ShapeDtypeStruct((B, A), jnp.float32),
        scratch_shapes=[pltpu.VMEM((B, D), jnp.float32),
                        pltpu.VMEM((B, D), jnp.float32)],
    )(acc2, h2s, dacc, b2r, batch3, state_vector, Wg, bgr, Wf, bfr)

    return out
